# baseline (speedup 1.0000x reference)
import functools
import jax
import jax.numpy as jnp
from jax import lax
from jax.experimental import pallas as pl
from jax.experimental.pallas import tpu as pltpu
from jax.experimental.pallas import tpu_sc as plsc

MAXR = 128


def _make_sc_kernel(L, D, rows):
    EXT = 2 * L
    NS = 16
    ext_per_tile = EXT // NS
    CH = 128
    n_ch = ext_per_tile // CH
    rows_per_worker = L // (2 * NS)

    mesh = plsc.VectorSubcoreMesh(core_axis_name="c", subcore_axis_name="s")

    @functools.partial(
        pl.kernel,
        mesh=mesh,
        out_type=jax.ShapeDtypeStruct((L, L, D), jnp.float32),
        compiler_params=pltpu.CompilerParams(use_tc_tiling_on_sc=False),
        scratch_types=[
            pltpu.VMEM_SHARED((EXT, D), jnp.float32),
            pltpu.VMEM((CH,), jnp.int32),
            pltpu.VMEM((CH, D), jnp.float32),
            pltpu.SemaphoreType.DMA,
        ],
    )
    def k(table_hbm, out_hbm, ext, idx, gbuf, sem):
        cid = lax.axis_index("c")
        sid = lax.axis_index("s")
        wid = sid * 2 + cid

        for c in range(n_ch):
            base = sid * ext_per_tile + c * CH
            for q in range(CH // 16):
                ii = lax.iota(jnp.int32, 16) + (base + q * 16 - (L - 1))
                idx[pl.ds(q * 16, 16)] = (
                    jnp.clip(ii, -MAXR, MAXR) + MAXR
                )
            pltpu.async_copy(table_hbm.at[idx], gbuf, sem).wait()
            pltpu.sync_copy(gbuf, ext.at[pl.ds(base, CH)])

        plsc.subcore_barrier()

        first = wid * rows_per_worker
        DEPTH = 8
        handles = []
        for t in range(rows_per_worker):
            i = first + t
            h = pltpu.async_copy(
                ext.at[pl.ds((L - 1) - i, L)], out_hbm.at[i], sem
            )
            handles.append(h)
            if t >= DEPTH:
                handles[t - DEPTH].wait()
        for t in range(rows_per_worker - DEPTH, rows_per_worker):
            handles[t].wait()

    return k


def kernel(time_x, length_q, embeddings_table):
    B, L, D = time_x.shape
    rows = embeddings_table.shape[0]
    out = _make_sc_kernel(L, D, rows)(embeddings_table)
    return jnp.broadcast_to(out[None], (B, L, L, D))


# speedup vs baseline: 3.1991x; 3.1991x over previous
import jax
import jax.numpy as jnp
from jax.experimental import pallas as pl
from jax.experimental.pallas import tpu as pltpu

MAXR = 128
LANES = 128


def _body(L, D, rows, tableT_ref, out_ref, ext_ref, rolled_ref):
    i0 = pl.program_id(0)
    t = pl.program_id(1)
    mid = L - MAXR

    @pl.when((i0 == 0) & (t == 0))
    def _build():
        ext_ref[:, 0:mid] = jnp.broadcast_to(tableT_ref[:, 0:1], (D, mid))
        ext_ref[:, mid:mid + rows - 1] = tableT_ref[:, 0:rows - 1]
        ext_ref[:, mid + rows - 1:2 * L] = jnp.broadcast_to(
            tableT_ref[:, rows - 1:rows], (D, 2 * L - mid - rows + 1)
        )

    s = jnp.where(i0 == 0, 0, LANES - i0)

    @pl.when(t == 0)
    def _roll():
        rolled_ref[:] = pltpu.roll(ext_ref[:], -s, axis=1)

    ntiles = L // LANES
    start = LANES * (ntiles - 1 - t) + jnp.where(s == 0, LANES, 0)
    start = pl.multiple_of(start, LANES)
    out_ref[0] = rolled_ref[:, pl.ds(start, L)]


def kernel(time_x, length_q, embeddings_table):
    B, L, D = time_x.shape
    rows = embeddings_table.shape[0]
    tableT = embeddings_table.T
    out_t = pl.pallas_call(
        lambda t, o, e, rl: _body(L, D, rows, t, o, e, rl),
        grid=(LANES, L // LANES),
        in_specs=[pl.BlockSpec((D, rows), lambda g0, g1: (0, 0))],
        out_specs=pl.BlockSpec(
            (1, D, L), lambda g0, g1: (g1 * LANES + g0, 0, 0)
        ),
        out_shape=jax.ShapeDtypeStruct((L, D, L), jnp.float32),
        scratch_shapes=[
            pltpu.VMEM((D, 2 * L), jnp.float32),
            pltpu.VMEM((D, 2 * L), jnp.float32),
        ],
    )(tableT)
    return jnp.broadcast_to(out_t[None].transpose(0, 1, 3, 2), (B, L, L, D))
